# baseline (speedup 1.0000x reference)
import functools

import jax
import jax.numpy as jnp
from jax import lax
from jax.experimental import pallas as pl
from jax.experimental.pallas import tpu as pltpu
from jax.experimental.pallas import tpu_sc as plsc

_N = 4019
_D = 64
_RB = 512
_NB = 8
_NPAD = _NB * _RB
_RBJ = 1024
_NBJ = _NPAD // _RBJ
_NB1 = 14
_K = 7
_BGATHER = _K * _NPAD


def _elu(x):
    return jnp.where(x > 0, x, jnp.exp(x) - 1.0)


def _k1_body(f_ref, w_ref, b_ref, g0_ref, g1_ref, ar_ref,
             h_ref, t0_ref, t1_ref, r0_ref):
    h = jnp.dot(f_ref[...], w_ref[...], preferred_element_type=jnp.float32)
    h = _elu(h + b_ref[...])
    h_ref[...] = h
    t0_ref[...] = jnp.dot(h, g0_ref[...], preferred_element_type=jnp.float32)
    t1_ref[...] = jnp.dot(h, g1_ref[...], preferred_element_type=jnp.float32)
    r0_ref[...] = jnp.dot(h, ar_ref[...], preferred_element_type=jnp.float32)


def _k1(feats_0, fc0_W, fc0_b, gcn0_W, gcn1_W, a_ref):
    kd = feats_0.shape[1]
    return pl.pallas_call(
        _k1_body,
        grid=(_NB,),
        in_specs=[
            pl.BlockSpec((_RB, kd), lambda i: (i, 0)),
            pl.BlockSpec((kd, _D), lambda i: (0, 0)),
            pl.BlockSpec((1, _D), lambda i: (0, 0)),
            pl.BlockSpec((_D, _D), lambda i: (0, 0)),
            pl.BlockSpec((_D, _D), lambda i: (0, 0)),
            pl.BlockSpec((_D, 1), lambda i: (0, 0)),
        ],
        out_specs=[
            pl.BlockSpec((_RB, _D), lambda i: (i, 0)),
            pl.BlockSpec((_RB, _D), lambda i: (i, 0)),
            pl.BlockSpec((_RB, _D), lambda i: (i, 0)),
            pl.BlockSpec((_RB, 1), lambda i: (i, 0)),
        ],
        out_shape=[
            jax.ShapeDtypeStruct((_N, _D), jnp.float32),
            jax.ShapeDtypeStruct((_N, _D), jnp.float32),
            jax.ShapeDtypeStruct((_N, _D), jnp.float32),
            jax.ShapeDtypeStruct((_N, 1), jnp.float32),
        ],
    )(feats_0, fc0_W, fc0_b, gcn0_W, gcn1_W, a_ref)


def _k2_body(f_ref, w_ref, b_ref, h_ref):
    h = jnp.dot(f_ref[...], w_ref[...], preferred_element_type=jnp.float32)
    h_ref[...] = _elu(h + b_ref[...])


def _k2(feats_1, fc1_W, fc1_b):
    v, kd = feats_1.shape
    return pl.pallas_call(
        _k2_body,
        grid=(_NB1,),
        in_specs=[
            pl.BlockSpec((_RB, kd), lambda i: (i, 0)),
            pl.BlockSpec((kd, _D), lambda i: (0, 0)),
            pl.BlockSpec((1, _D), lambda i: (0, 0)),
        ],
        out_specs=[pl.BlockSpec((_RB, _D), lambda i: (i, 0))],
        out_shape=[jax.ShapeDtypeStruct((v, _D), jnp.float32)],
    )(feats_1, fc1_W, fc1_b)[0]


def _sc_gather(table, flat_idx):
    info = plsc.get_sparse_core_info()
    nc, ns = info.num_cores, info.num_subcores
    nw = nc * ns
    b = flat_idx.shape[0]
    bpw = b // nw
    d = table.shape[1]
    mesh = plsc.VectorSubcoreMesh(core_axis_name="c", subcore_axis_name="s")

    @functools.partial(
        pl.kernel, mesh=mesh,
        out_type=jax.ShapeDtypeStruct((b, d), jnp.float32),
        scratch_types=[
            pltpu.VMEM((bpw,), jnp.int32),
            pltpu.VMEM((bpw, d), jnp.float32),
            pltpu.SemaphoreType.DMA,
        ],
        compiler_params=pltpu.CompilerParams(use_tc_tiling_on_sc=False),
    )
    def k(table_hbm, idx_hbm, out_hbm, idx_v, rows_v, sem):
        wid = lax.axis_index("s") * nc + lax.axis_index("c")
        base = wid * bpw
        pltpu.sync_copy(idx_hbm.at[pl.ds(base, bpw)], idx_v)
        pltpu.async_copy(table_hbm.at[idx_v], rows_v, sem).wait()
        pltpu.sync_copy(rows_v, out_hbm.at[pl.ds(base, bpw)])

    return k(table, flat_idx)


def _katt_body(rows_ref, h0_ref, ar_ref, an_ref, nei1_ref,
               f2_ref, w2_ref, b2_ref, e0_ref, e1_ref):
    rows3 = rows_ref[...]
    an = an_ref[...].reshape(1, 1, _D)
    s = jnp.sum(rows3 * an, axis=2)
    r0 = lax.dot_general(ar_ref[...], h0_ref[...],
                         (((1,), (1,)), ((), ())),
                         preferred_element_type=jnp.float32)
    att = s + r0
    att = jnp.where(att >= 0, att, 0.01 * att)
    att = att - jnp.max(att, axis=0, keepdims=True)
    w = jnp.exp(att)
    w = w / jnp.sum(w, axis=0, keepdims=True)
    emb0 = jnp.sum(rows3 * w[:, :, None], axis=0)
    e0_ref[...] = _elu(emb0)
    h2 = jnp.dot(f2_ref[...], w2_ref[...], preferred_element_type=jnp.float32)
    h2 = _elu(h2 + b2_ref[...])
    n2 = f2_ref.shape[0]
    oh = (nei1_ref[...] ==
          lax.broadcasted_iota(jnp.int32, (_RB, n2), 1)).astype(jnp.float32)
    e1_ref[...] = _elu(jnp.dot(oh, h2, preferred_element_type=jnp.float32))


def _katt(rows0, h0, a_ref_row, a_nei_row, nei1, feats_2, fc2_W, fc2_b):
    n2 = feats_2.shape[0]
    return pl.pallas_call(
        _katt_body,
        grid=(_NB,),
        in_specs=[
            pl.BlockSpec((_K, _RB, _D), lambda i: (0, i, 0)),
            pl.BlockSpec((_RB, _D), lambda i: (i, 0)),
            pl.BlockSpec((1, _D), lambda i: (0, 0)),
            pl.BlockSpec((1, _D), lambda i: (0, 0)),
            pl.BlockSpec((_RB, 1), lambda i: (i, 0)),
            pl.BlockSpec((n2, n2), lambda i: (0, 0)),
            pl.BlockSpec((n2, _D), lambda i: (0, 0)),
            pl.BlockSpec((1, _D), lambda i: (0, 0)),
        ],
        out_specs=[
            pl.BlockSpec((_RB, _D), lambda i: (i, 0)),
            pl.BlockSpec((_RB, _D), lambda i: (i, 0)),
        ],
        out_shape=[
            jax.ShapeDtypeStruct((_N, _D), jnp.float32),
            jax.ShapeDtypeStruct((_N, _D), jnp.float32),
        ],
    )(rows0, h0, a_ref_row, a_nei_row, nei1, feats_2, fc2_W, fc2_b)


def _k3_body(m_ref, t_ref, b_ref, p_ref, e_ref):
    a = jnp.dot(m_ref[...], t_ref[...],
                preferred_element_type=jnp.float32) + b_ref[...]
    e_ref[...] = jnp.where(a >= 0, a, p_ref[0, 0] * a)


def _k3(mps, t, gcn_b, prelu):
    rb3 = 1024
    return pl.pallas_call(
        _k3_body,
        grid=(_NPAD // rb3,),
        in_specs=[
            pl.BlockSpec((rb3, _N), lambda i: (i, 0)),
            pl.BlockSpec((_N, _D), lambda i: (0, 0)),
            pl.BlockSpec((1, _D), lambda i: (0, 0)),
            pl.BlockSpec((1, 1), lambda i: (0, 0)),
        ],
        out_specs=[pl.BlockSpec((rb3, _D), lambda i: (i, 0))],
        out_shape=[jax.ShapeDtypeStruct((_N, _D), jnp.float32)],
        compiler_params=pltpu.CompilerParams(
            vmem_limit_bytes=60 * 1024 * 1024),
    )(mps, t, gcn_b, prelu)[0]


def _k5_body(e0_ref, e1_ref, g0_ref, g1_ref, mW_ref, mb_ref, mv_ref,
             sW_ref, sb_ref, sv_ref, pW1_ref, pb1_ref, pW2_ref, pb2_ref,
             zn_ref, amp_ref, asc_ref):
    def sem(a, b, W, bb, v):
        m0 = jnp.mean(jnp.tanh(
            jnp.dot(a, W, preferred_element_type=jnp.float32) + bb),
            axis=0, keepdims=True)
        m1 = jnp.mean(jnp.tanh(
            jnp.dot(b, W, preferred_element_type=jnp.float32) + bb),
            axis=0, keepdims=True)
        s0 = jnp.sum(v * m0)
        s1 = jnp.sum(v * m1)
        mx = jnp.maximum(s0, s1)
        w0 = jnp.exp(s0 - mx)
        w1 = jnp.exp(s1 - mx)
        t = w0 + w1
        return (w0 / t) * a + (w1 / t) * b

    def proj(z, W1, b1, W2, b2):
        h = _elu(jnp.dot(z, W1, preferred_element_type=jnp.float32) + b1)
        return jnp.dot(h, W2, preferred_element_type=jnp.float32) + b2

    z_mp = sem(e0_ref[...], e1_ref[...], mW_ref[...], mb_ref[...], mv_ref[...])
    z_sc = sem(g0_ref[...], g1_ref[...], sW_ref[...], sb_ref[...], sv_ref[...])
    zp_mp = proj(z_mp, pW1_ref[...], pb1_ref[...], pW2_ref[...], pb2_ref[...])
    zp_sc = proj(z_sc, pW1_ref[...], pb1_ref[...], pW2_ref[...], pb2_ref[...])
    zn_ref[...] = z_mp / jnp.sqrt(jnp.sum(z_mp * z_mp, axis=1, keepdims=True))
    amp_ref[...] = zp_mp / jnp.sqrt(
        jnp.sum(zp_mp * zp_mp, axis=1, keepdims=True))
    asc_ref[...] = zp_sc / jnp.sqrt(
        jnp.sum(zp_sc * zp_sc, axis=1, keepdims=True))


def _k5(e0, e1, emb0, emb1, mW, mb, mv, sW, sb, sv, pW1, pb1, pW2, pb2):
    return pl.pallas_call(
        _k5_body,
        out_shape=[
            jax.ShapeDtypeStruct((_N, _D), jnp.float32),
            jax.ShapeDtypeStruct((_N, _D), jnp.float32),
            jax.ShapeDtypeStruct((_N, _D), jnp.float32),
        ],
    )(e0, e1, emb0, emb1, mW, mb, mv, sW, sb, sv, pW1, pb1, pW2, pb2)


def _k6_body(zni_ref, znj_ref, ampi_ref, ascj_ref, asci_ref, ampj_ref,
             pos_ref, pap_ref, psp_ref, loss_ref):
    i = pl.program_id(0)
    cdims = (((1,), (1,)), ((), ()))

    e = jnp.exp(lax.dot_general(zni_ref[...], znj_ref[...], cdims,
                                preferred_element_type=jnp.float32))
    s1 = jnp.exp(lax.dot_general(ampi_ref[...], ascj_ref[...], cdims,
                                 preferred_element_type=jnp.float32) * 1.25)
    s2 = jnp.exp(lax.dot_general(asci_ref[...], ampj_ref[...], cdims,
                                 preferred_element_type=jnp.float32) * 1.25)
    pos = pos_ref[...]
    pap = pap_ref[...]
    psp = psp_ref[...]

    rowid = lax.broadcasted_iota(jnp.int32, (_RB, 1), 0) + i * _RB
    colid = lax.broadcasted_iota(jnp.int32, (1, _N), 1)
    em = jnp.where(rowid == colid, 1.0, e)

    def rowsum(x):
        return jnp.sum(x, axis=1, keepdims=True)

    de = rowsum(em)
    npap = rowsum(em * pap)
    npsp = rowsum(em * psp)
    d1 = rowsum(s1)
    n1 = rowsum(s1 * pos)
    d2 = rowsum(s2)
    n2 = rowsum(s2 * pos)

    rmask = rowid < _N
    nc1 = jnp.sum(jnp.where(rmask, -jnp.log(npap / de + 1e-8), 0.0))
    nc2 = jnp.sum(jnp.where(rmask, -jnp.log(npsp / de + 1e-8), 0.0))
    lmp = jnp.sum(jnp.where(rmask, -jnp.log(n1 / (d1 + 1e-8)), 0.0))
    lsc = jnp.sum(jnp.where(rmask, -jnp.log(n2 / (d2 + 1e-8)), 0.0))
    part = (0.5 * lmp + 0.5 * lsc + nc1 + nc2) * (1.0 / _N)
    part = jnp.broadcast_to(part, (1, 1))

    @pl.when(i == 0)
    def _():
        loss_ref[...] = part

    @pl.when(i != 0)
    def _():
        loss_ref[...] += part


def _k6(zn, amp, asc, pos, pap1, psp1):
    blk_i = pl.BlockSpec((_RB, _D), lambda i: (i, 0))
    blk_a = pl.BlockSpec((_N, _D), lambda i: (0, 0))
    blk_m = pl.BlockSpec((_RB, _N), lambda i: (i, 0))
    return pl.pallas_call(
        _k6_body,
        grid=(_NB,),
        in_specs=[blk_i, blk_a, blk_i, blk_a, blk_i, blk_a,
                  blk_m, blk_m, blk_m],
        out_specs=[pl.BlockSpec((1, 1), lambda i: (0, 0))],
        out_shape=[jax.ShapeDtypeStruct((1, 1), jnp.float32)],
        compiler_params=pltpu.CompilerParams(
            vmem_limit_bytes=110 * 1024 * 1024),
    )(zn, zn, amp, asc, asc, amp, pos, pap1, psp1)[0]


def kernel(feats_0, feats_1, feats_2, pos, mps_0, mps_1, pap1, psp1,
           fc0_W, fc0_b, fc1_W, fc1_b, fc2_W, fc2_b,
           gcn0_W, gcn0_b, prelu0, gcn1_W, gcn1_b, prelu1,
           mp_att_fc_W, mp_att_fc_b, mp_att_vec, intra0_att, intra1_att,
           sc_att_fc_W, sc_att_fc_b, sc_att_vec,
           proj_W1, proj_b1, proj_W2, proj_b2, nei_index_0, nei_index_1):
    row = lambda v: v.reshape(1, _D)
    a_ref_row = intra0_att[:_D].reshape(1, _D)
    a_ref_col = intra0_att[:_D].reshape(_D, 1)
    a_nei_row = intra0_att[_D:].reshape(1, _D)

    h0, t0, t1, _r0 = _k1(feats_0, fc0_W, row(fc0_b), gcn0_W, gcn1_W,
                          a_ref_col)
    h1 = _k2(feats_1, fc1_W, row(fc1_b))

    ni0 = nei_index_0.astype(jnp.int32)
    ni0 = jnp.pad(ni0, ((0, _NPAD - _N), (0, 0)))
    flat_idx = ni0.T.reshape(-1)
    rows0 = _sc_gather(h1, flat_idx).reshape(_K, _NPAD, _D)

    e0 = _k3(mps_0, t0, row(gcn0_b), prelu0.reshape(1, 1))
    e1 = _k3(mps_1, t1, row(gcn1_b), prelu1.reshape(1, 1))

    ni1 = jnp.pad(nei_index_1.astype(jnp.int32), ((0, _NPAD - _N), (0, 0)))
    emb0, emb1 = _katt(rows0, h0, a_ref_row, a_nei_row, ni1,
                       feats_2, fc2_W, row(fc2_b))

    zn, amp, asc = _k5(e0, e1, emb0, emb1,
                       mp_att_fc_W, row(mp_att_fc_b), row(mp_att_vec),
                       sc_att_fc_W, row(sc_att_fc_b), row(sc_att_vec),
                       proj_W1, row(proj_b1), proj_W2, row(proj_b2))

    out = _k6(zn, amp, asc, pos, pap1, psp1)
    return out.reshape(())


# speedup vs baseline: 1.0129x; 1.0129x over previous
import functools

import jax
import jax.numpy as jnp
from jax import lax
from jax.experimental import pallas as pl
from jax.experimental.pallas import tpu as pltpu
from jax.experimental.pallas import tpu_sc as plsc

_N = 4019
_D = 64
_RB = 512
_NB = 8
_NPAD = _NB * _RB
_RBJ = 1024
_NBJ = _NPAD // _RBJ
_NB1 = 14
_K = 7
_BGATHER = _K * _NPAD


def _elu(x):
    return jnp.where(x > 0, x, jnp.exp(x) - 1.0)


def _k1_body(f_ref, w_ref, b_ref, g0_ref, g1_ref, ar_ref,
             h_ref, t0_ref, t1_ref, r0_ref):
    h = jnp.dot(f_ref[...], w_ref[...], preferred_element_type=jnp.float32)
    h = _elu(h + b_ref[...])
    h_ref[...] = h
    t0_ref[...] = jnp.dot(h, g0_ref[...], preferred_element_type=jnp.float32)
    t1_ref[...] = jnp.dot(h, g1_ref[...], preferred_element_type=jnp.float32)
    r0_ref[...] = jnp.dot(h, ar_ref[...], preferred_element_type=jnp.float32)


def _k1(feats_0, fc0_W, fc0_b, gcn0_W, gcn1_W, a_ref):
    kd = feats_0.shape[1]
    return pl.pallas_call(
        _k1_body,
        grid=(_NB,),
        in_specs=[
            pl.BlockSpec((_RB, kd), lambda i: (i, 0)),
            pl.BlockSpec((kd, _D), lambda i: (0, 0)),
            pl.BlockSpec((1, _D), lambda i: (0, 0)),
            pl.BlockSpec((_D, _D), lambda i: (0, 0)),
            pl.BlockSpec((_D, _D), lambda i: (0, 0)),
            pl.BlockSpec((_D, 1), lambda i: (0, 0)),
        ],
        out_specs=[
            pl.BlockSpec((_RB, _D), lambda i: (i, 0)),
            pl.BlockSpec((_RB, _D), lambda i: (i, 0)),
            pl.BlockSpec((_RB, _D), lambda i: (i, 0)),
            pl.BlockSpec((_RB, 1), lambda i: (i, 0)),
        ],
        out_shape=[
            jax.ShapeDtypeStruct((_N, _D), jnp.float32),
            jax.ShapeDtypeStruct((_N, _D), jnp.float32),
            jax.ShapeDtypeStruct((_N, _D), jnp.float32),
            jax.ShapeDtypeStruct((_N, 1), jnp.float32),
        ],
    )(feats_0, fc0_W, fc0_b, gcn0_W, gcn1_W, a_ref)


def _k2_body(f_ref, w_ref, b_ref, h_ref):
    h = jnp.dot(f_ref[...], w_ref[...], preferred_element_type=jnp.float32)
    h_ref[...] = _elu(h + b_ref[...])


def _k2(feats_1, fc1_W, fc1_b):
    v, kd = feats_1.shape
    rb2 = 768
    return pl.pallas_call(
        _k2_body,
        grid=(pl.cdiv(v, rb2),),
        in_specs=[
            pl.BlockSpec((rb2, kd), lambda i: (i, 0)),
            pl.BlockSpec((kd, _D), lambda i: (0, 0)),
            pl.BlockSpec((1, _D), lambda i: (0, 0)),
        ],
        out_specs=[pl.BlockSpec((rb2, _D), lambda i: (i, 0))],
        out_shape=[jax.ShapeDtypeStruct((v, _D), jnp.float32)],
        compiler_params=pltpu.CompilerParams(
            vmem_limit_bytes=60 * 1024 * 1024),
    )(feats_1, fc1_W, fc1_b)[0]


def _sc_gather(table, flat_idx):
    info = plsc.get_sparse_core_info()
    nc, ns = info.num_cores, info.num_subcores
    nw = nc * ns
    b = flat_idx.shape[0]
    bpw = b // nw
    d = table.shape[1]
    mesh = plsc.VectorSubcoreMesh(core_axis_name="c", subcore_axis_name="s")

    half = bpw // 2

    @functools.partial(
        pl.kernel, mesh=mesh,
        out_type=jax.ShapeDtypeStruct((b, d), jnp.float32),
        scratch_types=[
            pltpu.VMEM((bpw,), jnp.int32),
            pltpu.VMEM((bpw, d), jnp.float32),
            pltpu.SemaphoreType.DMA,
            pltpu.SemaphoreType.DMA,
        ],
        compiler_params=pltpu.CompilerParams(use_tc_tiling_on_sc=False),
    )
    def k(table_hbm, idx_hbm, out_hbm, idx_v, rows_v, sem, sem2):
        wid = lax.axis_index("s") * nc + lax.axis_index("c")
        base = wid * bpw
        pltpu.sync_copy(idx_hbm.at[pl.ds(base, bpw)], idx_v)
        c1 = pltpu.async_copy(table_hbm.at[idx_v.at[pl.ds(0, half)]],
                              rows_v.at[pl.ds(0, half)], sem)
        c2 = pltpu.async_copy(table_hbm.at[idx_v.at[pl.ds(half, half)]],
                              rows_v.at[pl.ds(half, half)], sem2)
        c1.wait()
        c2.wait()
        pltpu.sync_copy(rows_v, out_hbm.at[pl.ds(base, bpw)])

    return k(table, flat_idx)


def _katt_body(rows_ref, h0_ref, ar_ref, an_ref, nei1_ref,
               f2_ref, w2_ref, b2_ref, e0_ref, e1_ref):
    rows3 = rows_ref[...]
    an = an_ref[...].reshape(1, 1, _D)
    s = jnp.sum(rows3 * an, axis=2)
    r0 = lax.dot_general(ar_ref[...], h0_ref[...],
                         (((1,), (1,)), ((), ())),
                         preferred_element_type=jnp.float32)
    att = s + r0
    att = jnp.where(att >= 0, att, 0.01 * att)
    att = att - jnp.max(att, axis=0, keepdims=True)
    w = jnp.exp(att)
    w = w / jnp.sum(w, axis=0, keepdims=True)
    emb0 = jnp.sum(rows3 * w[:, :, None], axis=0)
    e0_ref[...] = _elu(emb0)
    h2 = jnp.dot(f2_ref[...], w2_ref[...], preferred_element_type=jnp.float32)
    h2 = _elu(h2 + b2_ref[...])
    n2 = f2_ref.shape[0]
    oh = (nei1_ref[...] ==
          lax.broadcasted_iota(jnp.int32, (_RB, n2), 1)).astype(jnp.float32)
    e1_ref[...] = _elu(jnp.dot(oh, h2, preferred_element_type=jnp.float32))


def _katt(rows0, h0, a_ref_row, a_nei_row, nei1, feats_2, fc2_W, fc2_b):
    n2 = feats_2.shape[0]
    return pl.pallas_call(
        _katt_body,
        grid=(_NB,),
        in_specs=[
            pl.BlockSpec((_K, _RB, _D), lambda i: (0, i, 0)),
            pl.BlockSpec((_RB, _D), lambda i: (i, 0)),
            pl.BlockSpec((1, _D), lambda i: (0, 0)),
            pl.BlockSpec((1, _D), lambda i: (0, 0)),
            pl.BlockSpec((_RB, 1), lambda i: (i, 0)),
            pl.BlockSpec((n2, n2), lambda i: (0, 0)),
            pl.BlockSpec((n2, _D), lambda i: (0, 0)),
            pl.BlockSpec((1, _D), lambda i: (0, 0)),
        ],
        out_specs=[
            pl.BlockSpec((_RB, _D), lambda i: (i, 0)),
            pl.BlockSpec((_RB, _D), lambda i: (i, 0)),
        ],
        out_shape=[
            jax.ShapeDtypeStruct((_N, _D), jnp.float32),
            jax.ShapeDtypeStruct((_N, _D), jnp.float32),
        ],
    )(rows0, h0, a_ref_row, a_nei_row, nei1, feats_2, fc2_W, fc2_b)


def _k3_body(m0_ref, m1_ref, t0_ref, t1_ref, b0_ref, b1_ref,
             p0_ref, p1_ref, e0_ref, e1_ref):
    a0 = jnp.dot(m0_ref[...], t0_ref[...],
                 preferred_element_type=jnp.float32) + b0_ref[...]
    e0_ref[...] = jnp.where(a0 >= 0, a0, p0_ref[0, 0] * a0)
    a1 = jnp.dot(m1_ref[...], t1_ref[...],
                 preferred_element_type=jnp.float32) + b1_ref[...]
    e1_ref[...] = jnp.where(a1 >= 0, a1, p1_ref[0, 0] * a1)


def _k3(mps_0, mps_1, t0, t1, gcn0_b, gcn1_b, prelu0, prelu1):
    return pl.pallas_call(
        _k3_body,
        grid=(_NB,),
        in_specs=[
            pl.BlockSpec((_RB, _N), lambda i: (i, 0)),
            pl.BlockSpec((_RB, _N), lambda i: (i, 0)),
            pl.BlockSpec((_N, _D), lambda i: (0, 0)),
            pl.BlockSpec((_N, _D), lambda i: (0, 0)),
            pl.BlockSpec((1, _D), lambda i: (0, 0)),
            pl.BlockSpec((1, _D), lambda i: (0, 0)),
            pl.BlockSpec((1, 1), lambda i: (0, 0)),
            pl.BlockSpec((1, 1), lambda i: (0, 0)),
        ],
        out_specs=[
            pl.BlockSpec((_RB, _D), lambda i: (i, 0)),
            pl.BlockSpec((_RB, _D), lambda i: (i, 0)),
        ],
        out_shape=[
            jax.ShapeDtypeStruct((_N, _D), jnp.float32),
            jax.ShapeDtypeStruct((_N, _D), jnp.float32),
        ],
    )(mps_0, mps_1, t0, t1, gcn0_b, gcn1_b, prelu0, prelu1)


def _k5_body(e0_ref, e1_ref, g0_ref, g1_ref, mW_ref, mb_ref, mv_ref,
             sW_ref, sb_ref, sv_ref, pW1_ref, pb1_ref, pW2_ref, pb2_ref,
             zn_ref, amp_ref, asc_ref):
    def sem(a, b, W, bb, v):
        m0 = jnp.mean(jnp.tanh(
            jnp.dot(a, W, preferred_element_type=jnp.float32) + bb),
            axis=0, keepdims=True)
        m1 = jnp.mean(jnp.tanh(
            jnp.dot(b, W, preferred_element_type=jnp.float32) + bb),
            axis=0, keepdims=True)
        s0 = jnp.sum(v * m0)
        s1 = jnp.sum(v * m1)
        mx = jnp.maximum(s0, s1)
        w0 = jnp.exp(s0 - mx)
        w1 = jnp.exp(s1 - mx)
        t = w0 + w1
        return (w0 / t) * a + (w1 / t) * b

    def proj(z, W1, b1, W2, b2):
        h = _elu(jnp.dot(z, W1, preferred_element_type=jnp.float32) + b1)
        return jnp.dot(h, W2, preferred_element_type=jnp.float32) + b2

    z_mp = sem(e0_ref[...], e1_ref[...], mW_ref[...], mb_ref[...], mv_ref[...])
    z_sc = sem(g0_ref[...], g1_ref[...], sW_ref[...], sb_ref[...], sv_ref[...])
    zp_mp = proj(z_mp, pW1_ref[...], pb1_ref[...], pW2_ref[...], pb2_ref[...])
    zp_sc = proj(z_sc, pW1_ref[...], pb1_ref[...], pW2_ref[...], pb2_ref[...])
    zn_ref[...] = z_mp / jnp.sqrt(jnp.sum(z_mp * z_mp, axis=1, keepdims=True))
    amp_ref[...] = zp_mp / jnp.sqrt(
        jnp.sum(zp_mp * zp_mp, axis=1, keepdims=True))
    asc_ref[...] = zp_sc / jnp.sqrt(
        jnp.sum(zp_sc * zp_sc, axis=1, keepdims=True))


def _k5(e0, e1, emb0, emb1, mW, mb, mv, sW, sb, sv, pW1, pb1, pW2, pb2):
    return pl.pallas_call(
        _k5_body,
        out_shape=[
            jax.ShapeDtypeStruct((_N, _D), jnp.float32),
            jax.ShapeDtypeStruct((_N, _D), jnp.float32),
            jax.ShapeDtypeStruct((_N, _D), jnp.float32),
        ],
    )(e0, e1, emb0, emb1, mW, mb, mv, sW, sb, sv, pW1, pb1, pW2, pb2)


def _k6_body(zni_ref, znj_ref, ampi_ref, ascj_ref, asci_ref, ampj_ref,
             pos_ref, pap_ref, psp_ref, loss_ref):
    i = pl.program_id(0)
    cdims = (((1,), (1,)), ((), ()))

    e = jnp.exp(lax.dot_general(zni_ref[...], znj_ref[...], cdims,
                                preferred_element_type=jnp.float32))
    s1 = jnp.exp(lax.dot_general(ampi_ref[...], ascj_ref[...], cdims,
                                 preferred_element_type=jnp.float32) * 1.25)
    s2 = jnp.exp(lax.dot_general(asci_ref[...], ampj_ref[...], cdims,
                                 preferred_element_type=jnp.float32) * 1.25)
    pos = pos_ref[...]
    pap = pap_ref[...]
    psp = psp_ref[...]

    rowid = lax.broadcasted_iota(jnp.int32, (_RB, 1), 0) + i * _RB
    colid = lax.broadcasted_iota(jnp.int32, (1, _N), 1)
    em = jnp.where(rowid == colid, 1.0, e)

    def rowsum(x):
        return jnp.sum(x, axis=1, keepdims=True)

    de = rowsum(em)
    npap = rowsum(em * pap)
    npsp = rowsum(em * psp)
    d1 = rowsum(s1)
    n1 = rowsum(s1 * pos)
    d2 = rowsum(s2)
    n2 = rowsum(s2 * pos)

    rmask = rowid < _N
    nc1 = jnp.sum(jnp.where(rmask, -jnp.log(npap / de + 1e-8), 0.0))
    nc2 = jnp.sum(jnp.where(rmask, -jnp.log(npsp / de + 1e-8), 0.0))
    lmp = jnp.sum(jnp.where(rmask, -jnp.log(n1 / (d1 + 1e-8)), 0.0))
    lsc = jnp.sum(jnp.where(rmask, -jnp.log(n2 / (d2 + 1e-8)), 0.0))
    part = (0.5 * lmp + 0.5 * lsc + nc1 + nc2) * (1.0 / _N)
    part = jnp.broadcast_to(part, (1, 1))

    @pl.when(i == 0)
    def _():
        loss_ref[...] = part

    @pl.when(i != 0)
    def _():
        loss_ref[...] += part


def _k6(zn, amp, asc, pos, pap1, psp1):
    blk_i = pl.BlockSpec((_RB, _D), lambda i: (i, 0))
    blk_a = pl.BlockSpec((_N, _D), lambda i: (0, 0))
    blk_m = pl.BlockSpec((_RB, _N), lambda i: (i, 0))
    return pl.pallas_call(
        _k6_body,
        grid=(_NB,),
        in_specs=[blk_i, blk_a, blk_i, blk_a, blk_i, blk_a,
                  blk_m, blk_m, blk_m],
        out_specs=[pl.BlockSpec((1, 1), lambda i: (0, 0))],
        out_shape=[jax.ShapeDtypeStruct((1, 1), jnp.float32)],
        compiler_params=pltpu.CompilerParams(
            vmem_limit_bytes=110 * 1024 * 1024),
    )(zn, zn, amp, asc, asc, amp, pos, pap1, psp1)[0]


def kernel(feats_0, feats_1, feats_2, pos, mps_0, mps_1, pap1, psp1,
           fc0_W, fc0_b, fc1_W, fc1_b, fc2_W, fc2_b,
           gcn0_W, gcn0_b, prelu0, gcn1_W, gcn1_b, prelu1,
           mp_att_fc_W, mp_att_fc_b, mp_att_vec, intra0_att, intra1_att,
           sc_att_fc_W, sc_att_fc_b, sc_att_vec,
           proj_W1, proj_b1, proj_W2, proj_b2, nei_index_0, nei_index_1):
    row = lambda v: v.reshape(1, _D)
    a_ref_row = intra0_att[:_D].reshape(1, _D)
    a_ref_col = intra0_att[:_D].reshape(_D, 1)
    a_nei_row = intra0_att[_D:].reshape(1, _D)

    h0, t0, t1, _r0 = _k1(feats_0, fc0_W, row(fc0_b), gcn0_W, gcn1_W,
                          a_ref_col)
    h1 = _k2(feats_1, fc1_W, row(fc1_b))

    ni0 = nei_index_0.astype(jnp.int32)
    ni0 = jnp.pad(ni0, ((0, _NPAD - _N), (0, 0)))
    flat_idx = ni0.T.reshape(-1)
    rows0 = _sc_gather(h1, flat_idx).reshape(_K, _NPAD, _D)

    e0, e1 = _k3(mps_0, mps_1, t0, t1, row(gcn0_b), row(gcn1_b),
                 prelu0.reshape(1, 1), prelu1.reshape(1, 1))

    ni1 = jnp.pad(nei_index_1.astype(jnp.int32), ((0, _NPAD - _N), (0, 0)))
    emb0, emb1 = _katt(rows0, h0, a_ref_row, a_nei_row, ni1,
                       feats_2, fc2_W, row(fc2_b))

    zn, amp, asc = _k5(e0, e1, emb0, emb1,
                       mp_att_fc_W, row(mp_att_fc_b), row(mp_att_vec),
                       sc_att_fc_W, row(sc_att_fc_b), row(sc_att_vec),
                       proj_W1, row(proj_b1), proj_W2, row(proj_b2))

    out = _k6(zn, amp, asc, pos, pap1, psp1)
    return out.reshape(())


# speedup vs baseline: 1.0229x; 1.0099x over previous
import functools

import jax
import jax.numpy as jnp
from jax import lax
from jax.experimental import pallas as pl
from jax.experimental.pallas import tpu as pltpu
from jax.experimental.pallas import tpu_sc as plsc

_N = 4019
_D = 64
_RB = 512
_NB = 8
_NPAD = _NB * _RB
_NB1 = 14
_K = 7


def _elu(x):
    return jnp.where(x > 0, x, jnp.exp(x) - 1.0)


def _k1_body(f_ref, w_ref, b_ref, g0_ref, g1_ref,
             h_ref, t0_ref, t1_ref):
    h = jnp.dot(f_ref[...], w_ref[...], preferred_element_type=jnp.float32)
    h = _elu(h + b_ref[...])
    h_ref[...] = h
    t0_ref[...] = jnp.dot(h, g0_ref[...], preferred_element_type=jnp.float32)
    t1_ref[...] = jnp.dot(h, g1_ref[...], preferred_element_type=jnp.float32)


def _k1(feats_0, fc0_W, fc0_b, gcn0_W, gcn1_W):
    kd = feats_0.shape[1]
    return pl.pallas_call(
        _k1_body,
        grid=(_NB,),
        in_specs=[
            pl.BlockSpec((_RB, kd), lambda i: (i, 0)),
            pl.BlockSpec((kd, _D), lambda i: (0, 0)),
            pl.BlockSpec((1, _D), lambda i: (0, 0)),
            pl.BlockSpec((_D, _D), lambda i: (0, 0)),
            pl.BlockSpec((_D, _D), lambda i: (0, 0)),
        ],
        out_specs=[
            pl.BlockSpec((_RB, _D), lambda i: (i, 0)),
            pl.BlockSpec((_RB, _D), lambda i: (i, 0)),
            pl.BlockSpec((_RB, _D), lambda i: (i, 0)),
        ],
        out_shape=[
            jax.ShapeDtypeStruct((_N, _D), jnp.float32),
            jax.ShapeDtypeStruct((_N, _D), jnp.float32),
            jax.ShapeDtypeStruct((_N, _D), jnp.float32),
        ],
    )(feats_0, fc0_W, fc0_b, gcn0_W, gcn1_W)


def _k2_body(f_ref, w_ref, b_ref, h_ref):
    h = jnp.dot(f_ref[...], w_ref[...], preferred_element_type=jnp.float32)
    h_ref[...] = _elu(h + b_ref[...])


def _k2(feats_1, fc1_W, fc1_b):
    v, kd = feats_1.shape
    return pl.pallas_call(
        _k2_body,
        grid=(_NB1,),
        in_specs=[
            pl.BlockSpec((_RB, kd), lambda i: (i, 0)),
            pl.BlockSpec((kd, _D), lambda i: (0, 0)),
            pl.BlockSpec((1, _D), lambda i: (0, 0)),
        ],
        out_specs=[pl.BlockSpec((_RB, _D), lambda i: (i, 0))],
        out_shape=[jax.ShapeDtypeStruct((v, _D), jnp.float32)],
    )(feats_1, fc1_W, fc1_b)[0]


def _sc_gather(table, flat_idx):
    info = plsc.get_sparse_core_info()
    nc, ns = info.num_cores, info.num_subcores
    nw = nc * ns
    b = flat_idx.shape[0]
    bpw = b // nw
    d = table.shape[1]
    mesh = plsc.VectorSubcoreMesh(core_axis_name="c", subcore_axis_name="s")

    @functools.partial(
        pl.kernel, mesh=mesh,
        out_type=jax.ShapeDtypeStruct((b, d), jnp.float32),
        scratch_types=[
            pltpu.VMEM((bpw,), jnp.int32),
            pltpu.VMEM((bpw, d), jnp.float32),
            pltpu.SemaphoreType.DMA,
        ],
        compiler_params=pltpu.CompilerParams(use_tc_tiling_on_sc=False),
    )
    def k(table_hbm, idx_hbm, out_hbm, idx_v, rows_v, sem):
        wid = lax.axis_index("s") * nc + lax.axis_index("c")
        base = wid * bpw
        pltpu.sync_copy(idx_hbm.at[pl.ds(base, bpw)], idx_v)
        pltpu.async_copy(table_hbm.at[idx_v], rows_v, sem).wait()
        pltpu.sync_copy(rows_v, out_hbm.at[pl.ds(base, bpw)])

    return k(table, flat_idx)


def _katt_body(rows_ref, h0_ref, ar_ref, an_ref, nei1_ref,
               f2_ref, w2_ref, b2_ref, e0_ref, e1_ref):
    rows3 = rows_ref[...]
    an = an_ref[...].reshape(1, 1, _D)
    s = jnp.sum(rows3 * an, axis=2)
    r0 = lax.dot_general(ar_ref[...], h0_ref[...],
                         (((1,), (1,)), ((), ())),
                         preferred_element_type=jnp.float32)
    att = s + r0
    att = jnp.where(att >= 0, att, 0.01 * att)
    att = att - jnp.max(att, axis=0, keepdims=True)
    w = jnp.exp(att)
    w = w / jnp.sum(w, axis=0, keepdims=True)
    emb0 = jnp.sum(rows3 * w[:, :, None], axis=0)
    e0_ref[...] = _elu(emb0)
    h2 = jnp.dot(f2_ref[...], w2_ref[...], preferred_element_type=jnp.float32)
    h2 = _elu(h2 + b2_ref[...])
    n2 = f2_ref.shape[0]
    oh = (nei1_ref[...] ==
          lax.broadcasted_iota(jnp.int32, (_RB, n2), 1)).astype(jnp.float32)
    e1_ref[...] = _elu(jnp.dot(oh, h2, preferred_element_type=jnp.float32))


def _katt(rows0, h0, a_ref_row, a_nei_row, nei1, feats_2, fc2_W, fc2_b):
    n2 = feats_2.shape[0]
    return pl.pallas_call(
        _katt_body,
        grid=(_NB,),
        in_specs=[
            pl.BlockSpec((_K, _RB, _D), lambda i: (0, i, 0)),
            pl.BlockSpec((_RB, _D), lambda i: (i, 0)),
            pl.BlockSpec((1, _D), lambda i: (0, 0)),
            pl.BlockSpec((1, _D), lambda i: (0, 0)),
            pl.BlockSpec((_RB, 1), lambda i: (i, 0)),
            pl.BlockSpec((n2, n2), lambda i: (0, 0)),
            pl.BlockSpec((n2, _D), lambda i: (0, 0)),
            pl.BlockSpec((1, _D), lambda i: (0, 0)),
        ],
        out_specs=[
            pl.BlockSpec((_RB, _D), lambda i: (i, 0)),
            pl.BlockSpec((_RB, _D), lambda i: (i, 0)),
        ],
        out_shape=[
            jax.ShapeDtypeStruct((_N, _D), jnp.float32),
            jax.ShapeDtypeStruct((_N, _D), jnp.float32),
        ],
    )(rows0, h0, a_ref_row, a_nei_row, nei1, feats_2, fc2_W, fc2_b)


def _k3_body(m0_ref, m1_ref, t0_ref, t1_ref, b0_ref, b1_ref,
             p0_ref, p1_ref, e0_ref, e1_ref):
    a0 = jnp.dot(m0_ref[...], t0_ref[...],
                 preferred_element_type=jnp.float32) + b0_ref[...]
    e0_ref[...] = jnp.where(a0 >= 0, a0, p0_ref[0, 0] * a0)
    a1 = jnp.dot(m1_ref[...], t1_ref[...],
                 preferred_element_type=jnp.float32) + b1_ref[...]
    e1_ref[...] = jnp.where(a1 >= 0, a1, p1_ref[0, 0] * a1)


def _k3(mps_0, mps_1, t0, t1, gcn0_b, gcn1_b, prelu0, prelu1):
    return pl.pallas_call(
        _k3_body,
        grid=(_NB,),
        in_specs=[
            pl.BlockSpec((_RB, _N), lambda i: (i, 0)),
            pl.BlockSpec((_RB, _N), lambda i: (i, 0)),
            pl.BlockSpec((_N, _D), lambda i: (0, 0)),
            pl.BlockSpec((_N, _D), lambda i: (0, 0)),
            pl.BlockSpec((1, _D), lambda i: (0, 0)),
            pl.BlockSpec((1, _D), lambda i: (0, 0)),
            pl.BlockSpec((1, 1), lambda i: (0, 0)),
            pl.BlockSpec((1, 1), lambda i: (0, 0)),
        ],
        out_specs=[
            pl.BlockSpec((_RB, _D), lambda i: (i, 0)),
            pl.BlockSpec((_RB, _D), lambda i: (i, 0)),
        ],
        out_shape=[
            jax.ShapeDtypeStruct((_N, _D), jnp.float32),
            jax.ShapeDtypeStruct((_N, _D), jnp.float32),
        ],
    )(mps_0, mps_1, t0, t1, gcn0_b, gcn1_b, prelu0, prelu1)


def _k5_body(e0_ref, e1_ref, g0_ref, g1_ref, mW_ref, mb_ref, mv_ref,
             sW_ref, sb_ref, sv_ref, pW1_ref, pb1_ref, pW2_ref, pb2_ref,
             zn_ref, amp_ref, asc_ref):
    def sem(a, b, W, bb, v):
        m0 = jnp.mean(jnp.tanh(
            jnp.dot(a, W, preferred_element_type=jnp.float32) + bb),
            axis=0, keepdims=True)
        m1 = jnp.mean(jnp.tanh(
            jnp.dot(b, W, preferred_element_type=jnp.float32) + bb),
            axis=0, keepdims=True)
        s0 = jnp.sum(v * m0)
        s1 = jnp.sum(v * m1)
        mx = jnp.maximum(s0, s1)
        w0 = jnp.exp(s0 - mx)
        w1 = jnp.exp(s1 - mx)
        t = w0 + w1
        return (w0 / t) * a + (w1 / t) * b

    def proj(z, W1, b1, W2, b2):
        h = _elu(jnp.dot(z, W1, preferred_element_type=jnp.float32) + b1)
        return jnp.dot(h, W2, preferred_element_type=jnp.float32) + b2

    z_mp = sem(e0_ref[...], e1_ref[...], mW_ref[...], mb_ref[...], mv_ref[...])
    z_sc = sem(g0_ref[...], g1_ref[...], sW_ref[...], sb_ref[...], sv_ref[...])
    zp_mp = proj(z_mp, pW1_ref[...], pb1_ref[...], pW2_ref[...], pb2_ref[...])
    zp_sc = proj(z_sc, pW1_ref[...], pb1_ref[...], pW2_ref[...], pb2_ref[...])
    zn_ref[...] = z_mp / jnp.sqrt(jnp.sum(z_mp * z_mp, axis=1, keepdims=True))
    amp_ref[...] = zp_mp / jnp.sqrt(
        jnp.sum(zp_mp * zp_mp, axis=1, keepdims=True))
    asc_ref[...] = zp_sc / jnp.sqrt(
        jnp.sum(zp_sc * zp_sc, axis=1, keepdims=True))


def _k5(e0, e1, emb0, emb1, mW, mb, mv, sW, sb, sv, pW1, pb1, pW2, pb2):
    return pl.pallas_call(
        _k5_body,
        out_shape=[
            jax.ShapeDtypeStruct((_N, _D), jnp.float32),
            jax.ShapeDtypeStruct((_N, _D), jnp.float32),
            jax.ShapeDtypeStruct((_N, _D), jnp.float32),
        ],
    )(e0, e1, emb0, emb1, mW, mb, mv, sW, sb, sv, pW1, pb1, pW2, pb2)


def _k6_body(zni_ref, znj_ref, ampi_ref, ascj_ref, asci_ref, ampj_ref,
             pos_ref, pap_ref, psp_ref, loss_ref):
    i = pl.program_id(0)
    cdims = (((1,), (1,)), ((), ()))

    e = jnp.exp(lax.dot_general(zni_ref[...], znj_ref[...], cdims,
                                preferred_element_type=jnp.float32))
    s1 = jnp.exp(lax.dot_general(ampi_ref[...], ascj_ref[...], cdims,
                                 preferred_element_type=jnp.float32) * 1.25)
    s2 = jnp.exp(lax.dot_general(asci_ref[...], ampj_ref[...], cdims,
                                 preferred_element_type=jnp.float32) * 1.25)
    pos = pos_ref[...]
    pap = pap_ref[...]
    psp = psp_ref[...]

    rowid = lax.broadcasted_iota(jnp.int32, (_RB, 1), 0) + i * _RB
    colid = lax.broadcasted_iota(jnp.int32, (1, _N), 1)
    em = jnp.where(rowid == colid, 1.0, e)

    def rowsum(x):
        return jnp.sum(x, axis=1, keepdims=True)

    de = rowsum(em)
    npap = rowsum(em * pap)
    npsp = rowsum(em * psp)
    d1 = rowsum(s1)
    n1 = rowsum(s1 * pos)
    d2 = rowsum(s2)
    n2 = rowsum(s2 * pos)

    rmask = rowid < _N
    nc1 = jnp.sum(jnp.where(rmask, -jnp.log(npap / de + 1e-8), 0.0))
    nc2 = jnp.sum(jnp.where(rmask, -jnp.log(npsp / de + 1e-8), 0.0))
    lmp = jnp.sum(jnp.where(rmask, -jnp.log(n1 / (d1 + 1e-8)), 0.0))
    lsc = jnp.sum(jnp.where(rmask, -jnp.log(n2 / (d2 + 1e-8)), 0.0))
    part = (0.5 * lmp + 0.5 * lsc + nc1 + nc2) * (1.0 / _N)
    part = jnp.broadcast_to(part, (1, 1))

    @pl.when(i == 0)
    def _():
        loss_ref[...] = part

    @pl.when(i != 0)
    def _():
        loss_ref[...] += part


def _k6(zn, amp, asc, pos, pap1, psp1):
    blk_i = pl.BlockSpec((_RB, _D), lambda i: (i, 0))
    blk_a = pl.BlockSpec((_N, _D), lambda i: (0, 0))
    blk_m = pl.BlockSpec((_RB, _N), lambda i: (i, 0))
    return pl.pallas_call(
        _k6_body,
        grid=(_NB,),
        in_specs=[blk_i, blk_a, blk_i, blk_a, blk_i, blk_a,
                  blk_m, blk_m, blk_m],
        out_specs=[pl.BlockSpec((1, 1), lambda i: (0, 0))],
        out_shape=[jax.ShapeDtypeStruct((1, 1), jnp.float32)],
        compiler_params=pltpu.CompilerParams(
            vmem_limit_bytes=110 * 1024 * 1024),
    )(zn, zn, amp, asc, asc, amp, pos, pap1, psp1)[0]


def kernel(feats_0, feats_1, feats_2, pos, mps_0, mps_1, pap1, psp1,
           fc0_W, fc0_b, fc1_W, fc1_b, fc2_W, fc2_b,
           gcn0_W, gcn0_b, prelu0, gcn1_W, gcn1_b, prelu1,
           mp_att_fc_W, mp_att_fc_b, mp_att_vec, intra0_att, intra1_att,
           sc_att_fc_W, sc_att_fc_b, sc_att_vec,
           proj_W1, proj_b1, proj_W2, proj_b2, nei_index_0, nei_index_1):
    row = lambda v: v.reshape(1, _D)
    a_ref_row = intra0_att[:_D].reshape(1, _D)
    a_nei_row = intra0_att[_D:].reshape(1, _D)

    h0, t0, t1 = _k1(feats_0, fc0_W, row(fc0_b), gcn0_W, gcn1_W)
    h1 = _k2(feats_1, fc1_W, row(fc1_b))

    ni0 = nei_index_0.astype(jnp.int32)
    ni0 = jnp.pad(ni0, ((0, _NPAD - _N), (0, 0)))
    flat_idx = ni0.T.reshape(-1)
    rows0 = _sc_gather(h1, flat_idx).reshape(_K, _NPAD, _D)

    e0, e1 = _k3(mps_0, mps_1, t0, t1, row(gcn0_b), row(gcn1_b),
                 prelu0.reshape(1, 1), prelu1.reshape(1, 1))

    ni1 = jnp.pad(nei_index_1.astype(jnp.int32), ((0, _NPAD - _N), (0, 0)))
    emb0, emb1 = _katt(rows0, h0, a_ref_row, a_nei_row, ni1,
                       feats_2, fc2_W, row(fc2_b))

    zn, amp, asc = _k5(e0, e1, emb0, emb1,
                       mp_att_fc_W, row(mp_att_fc_b), row(mp_att_vec),
                       sc_att_fc_W, row(sc_att_fc_b), row(sc_att_vec),
                       proj_W1, row(proj_b1), proj_W2, row(proj_b2))

    out = _k6(zn, amp, asc, pos, pap1, psp1)
    return out.reshape(())
